# Initial kernel scaffold; baseline (speedup 1.0000x reference)
import jax
import jax.numpy as jnp
from jax.experimental import pallas as pl


def kernel(x, indices):
    raise NotImplementedError("write your pallas kernel here")


# speedup vs baseline: 4.8586x; 4.8586x over previous
import jax
import jax.numpy as jnp
from jax.experimental import pallas as pl
from jax.experimental.pallas import tpu as pltpu

_ROWS_PER_BLOCK = 2048


def _gather_block(x_ref, idx_ref, out_ref):
    idx = idx_ref[0, :]
    col = jax.lax.broadcasted_iota(jnp.int32, (512, 128), 0)
    onehot = (col == idx[None, :]).astype(jnp.float32)
    out_ref[...] = jnp.dot(x_ref[...], onehot,
                           preferred_element_type=jnp.float32)


def kernel(x, indices):
    n_rows, n_cols = x.shape
    k = indices.shape[0]
    grid = (n_rows // _ROWS_PER_BLOCK,)
    return pl.pallas_call(
        _gather_block,
        grid=grid,
        in_specs=[
            pl.BlockSpec((_ROWS_PER_BLOCK, n_cols), lambda i: (i, 0)),
            pl.BlockSpec((1, k), lambda i: (0, 0)),
        ],
        out_specs=pl.BlockSpec((_ROWS_PER_BLOCK, k), lambda i: (i, 0)),
        out_shape=jax.ShapeDtypeStruct((n_rows, k), jnp.float32),
    )(x, indices.reshape(1, k))
